# baseline (speedup 1.0000x reference)
import functools
import math

import jax
import jax.numpy as jnp
from jax import lax
from jax.experimental import pallas as pl
from jax.experimental.pallas import tpu as pltpu
from jax.experimental.pallas import tpu_sc as plsc


def _stage_a(x, g_w1T, g_b1, g_w2T, g_b2, cc_mat, tn=2048):
    B, N, C = x.shape
    CR = g_w1T.shape[1]

    def body(x_ref, w1_ref, b1_ref, w2_ref, b2_ref, cc_ref, g_ref, m_ref):
        xs = x_ref[0]
        h = jnp.dot(xs, w1_ref[...], preferred_element_type=jnp.float32) + b1_ref[...]
        h = jnp.maximum(h, 0.0)
        z = jnp.dot(h, w2_ref[...], preferred_element_type=jnp.float32) + b2_ref[...]
        g_ref[0] = 1.0 / (1.0 + jnp.exp(-z))
        m_ref[0] = jnp.dot(xs, cc_ref[...], preferred_element_type=jnp.float32)

    return pl.pallas_call(
        body,
        grid=(B, N // tn),
        in_specs=[
            pl.BlockSpec((1, tn, C), lambda b, i: (b, i, 0)),
            pl.BlockSpec((C, CR), lambda b, i: (0, 0)),
            pl.BlockSpec((1, CR), lambda b, i: (0, 0)),
            pl.BlockSpec((CR, 1), lambda b, i: (0, 0)),
            pl.BlockSpec((1, 1), lambda b, i: (0, 0)),
            pl.BlockSpec((C, 9), lambda b, i: (0, 0)),
        ],
        out_specs=[
            pl.BlockSpec((1, tn, 1), lambda b, i: (b, i, 0)),
            pl.BlockSpec((1, tn, 9), lambda b, i: (b, i, 0)),
        ],
        out_shape=[
            jax.ShapeDtypeStruct((B, N, 1), jnp.float32),
            jax.ShapeDtypeStruct((B, N, 9), jnp.float32),
        ],
    )(x, g_w1T, g_b1, g_w2T, g_b2, cc_mat)


def _stage_b(M9, gate_img, SH, SHT, K):
    B, _, Hc, Wc = M9.shape
    N = Hc * Wc

    def body(m_ref, g_ref, sh_ref, sht_ref, idx_ref, sv_ref, cv_ref):
        y = jnp.zeros((Hc, Wc), jnp.float32)
        for di in range(3):
            acc = jnp.zeros((Hc, Wc), jnp.float32)
            for dj in range(3):
                acc = acc + jnp.dot(m_ref[0, di * 3 + dj], sht_ref[dj],
                                    preferred_element_type=jnp.float32)
            y = y + jnp.dot(sh_ref[di], acc, preferred_element_type=jnp.float32)
        s1 = jnp.sum(y, axis=1, keepdims=True)
        mu = jnp.sum(s1, axis=0, keepdims=True) / N
        d = y - mu
        s2 = jnp.sum(d * d, axis=1, keepdims=True)
        var = jnp.sum(s2, axis=0, keepdims=True) / N
        curv = d / jnp.sqrt(var + 1e-5)
        score = (jnp.abs(curv) + g_ref[0]) * 0.5

        rowi = lax.broadcasted_iota(jnp.int32, (Hc, Wc), 0)
        coli = lax.broadcasted_iota(jnp.int32, (Hc, Wc), 1)
        I2 = (rowi * Wc + coli).astype(jnp.float32)
        lane = lax.broadcasted_iota(jnp.int32, (1, K), 1).astype(jnp.float32)

        def it(i, carry):
            sc, ia, va, ca = carry
            mr = jnp.max(sc, axis=1, keepdims=True)
            m = jnp.max(mr, axis=0, keepdims=True)
            cand = jnp.where(sc == m, I2, jnp.float32(1e9))
            ir = jnp.min(cand, axis=1, keepdims=True)
            idx = jnp.min(ir, axis=0, keepdims=True)
            hit = I2 == idx
            cr = jnp.sum(jnp.where(hit, curv, 0.0), axis=1, keepdims=True)
            cval = jnp.sum(cr, axis=0, keepdims=True)
            fi = lane == i.astype(jnp.float32)
            ia = jnp.where(fi, idx, ia)
            va = jnp.where(fi, m, va)
            ca = jnp.where(fi, cval, ca)
            sc = jnp.where(hit, jnp.float32(-3e38), sc)
            return sc, ia, va, ca

        z = jnp.zeros((1, K), jnp.float32)
        _, ia, va, ca = lax.fori_loop(0, K, it, (score, z, z, z))
        b = pl.program_id(0)
        ia = ia + b.astype(jnp.float32) * N
        idx_ref[0] = ia.astype(jnp.int32)
        sv_ref[0] = va
        cv_ref[0] = ca

    return pl.pallas_call(
        body,
        grid=(B,),
        in_specs=[
            pl.BlockSpec((1, 9, Hc, Wc), lambda b: (b, 0, 0, 0)),
            pl.BlockSpec((1, Hc, Wc), lambda b: (b, 0, 0)),
            pl.BlockSpec((3, Hc, Hc), lambda b: (0, 0, 0)),
            pl.BlockSpec((3, Wc, Wc), lambda b: (0, 0, 0)),
        ],
        out_specs=[
            pl.BlockSpec((1, 1, K), lambda b: (b, 0, 0)),
            pl.BlockSpec((1, 1, K), lambda b: (b, 0, 0)),
            pl.BlockSpec((1, 1, K), lambda b: (b, 0, 0)),
        ],
        out_shape=[
            jax.ShapeDtypeStruct((B, 1, K), jnp.int32),
            jax.ShapeDtypeStruct((B, 1, K), jnp.float32),
            jax.ShapeDtypeStruct((B, 1, K), jnp.float32),
        ],
    )(M9, gate_img, SH, SHT)


def _sc_gather(x2d, gidx):
    R = gidx.shape[0]
    C = x2d.shape[1]
    rpw = 8
    nw = R // rpw
    mesh = plsc.VectorSubcoreMesh(core_axis_name="c", subcore_axis_name="s")

    @functools.partial(
        pl.kernel,
        mesh=mesh,
        out_type=jax.ShapeDtypeStruct((R, C), jnp.float32),
        scratch_types=[
            pltpu.VMEM((rpw,), jnp.int32),
            pltpu.VMEM((rpw, C), jnp.float32),
            pltpu.SemaphoreType.DMA,
        ],
    )
    def k(x_hbm, idx_hbm, out_hbm, idx_v, rows_v, sem):
        wid = lax.axis_index("s") * 2 + lax.axis_index("c")

        @pl.when(wid < nw)
        def _():
            base = wid * rpw
            pltpu.sync_copy(idx_hbm.at[pl.ds(base, rpw)], idx_v)
            pltpu.async_copy(x_hbm.at[idx_v], rows_v, sem).wait()
            pltpu.sync_copy(rows_v, out_hbm.at[pl.ds(base, rpw)])

    return k(x2d, gidx)


def _erf(z):
    a1, a2, a3, a4, a5 = 0.254829592, -0.284496736, 1.421413741, -1.453152027, 1.061405429
    p = 0.3275911
    s = jnp.sign(z)
    az = jnp.abs(z)
    t = 1.0 / (1.0 + p * az)
    poly = t * (a1 + t * (a2 + t * (a3 + t * (a4 + t * a5))))
    return s * (1.0 - poly * jnp.exp(-az * az))


def _stage_c(xk, s_top, c_top, alpha2, kv_wT, kv_b2, ln_g2, ln_b2, k_wT, v_wT,
             cpe_mat, cpe_b2, q_w, proj_wT, SH64, kmaskT, vmask, eyeK,
             qk_scale, NH, K):
    B, _, C = xk.shape
    CR = kv_wT.shape[1]
    KB = NH * K

    def body(xk_ref, s_ref, c_ref, a_ref, kvw_ref, kvb_ref, lng_ref, lnb_ref,
             kw_ref, vw_ref, cpem_ref, cpeb_ref, qw_ref, pw_ref, sh_ref,
             km_ref, vm_ref, eye_ref, qkt_ref, vp_ref, cm_ref):
        xs = xk_ref[0]
        kv = jnp.dot(xs, kvw_ref[...], preferred_element_type=jnp.float32) + kvb_ref[...]
        mu = jnp.mean(kv, axis=1, keepdims=True)
        d = kv - mu
        var = jnp.mean(d * d, axis=1, keepdims=True)
        kv = d / jnp.sqrt(var + 1e-5) * lng_ref[...] + lnb_ref[...]
        kv = 0.5 * kv * (1.0 + _erf(kv * (2.0 ** -0.5)))
        k = jnp.dot(kv, kw_ref[...], preferred_element_type=jnp.float32)
        v = jnp.dot(kv, vw_ref[...], preferred_element_type=jnp.float32)
        srow = s_ref[0]
        scol = jnp.sum(eye_ref[...] * srow, axis=1, keepdims=True)
        v = v * scol
        cpem = cpem_ref[...]
        vc = jnp.zeros_like(v)
        for t in range(9):
            vc = vc + jnp.dot(sh_ref[t], v,
                              preferred_element_type=jnp.float32) * cpem[t:t + 1, :]
        v = v + vc + cpeb_ref[...]
        kbt = jnp.concatenate([k] * NH, axis=0) * km_ref[...]
        qkt_ref[0] = jnp.dot(kbt, qw_ref[...],
                             preferred_element_type=jnp.float32) * qk_scale
        vb = jnp.concatenate([v] * NH, axis=0) * vm_ref[...]
        vp_ref[0] = jnp.dot(vb, pw_ref[...], preferred_element_type=jnp.float32)
        crow = c_ref[0]
        cm_ref[0] = jnp.concatenate([crow] * NH, axis=1) * a_ref[0, 0]

    return pl.pallas_call(
        body,
        grid=(B,),
        in_specs=[
            pl.BlockSpec((1, K, C), lambda b: (b, 0, 0)),
            pl.BlockSpec((1, 1, K), lambda b: (b, 0, 0)),
            pl.BlockSpec((1, 1, K), lambda b: (b, 0, 0)),
            pl.BlockSpec((1, 1), lambda b: (0, 0)),
            pl.BlockSpec((C, CR), lambda b: (0, 0)),
            pl.BlockSpec((1, CR), lambda b: (0, 0)),
            pl.BlockSpec((1, CR), lambda b: (0, 0)),
            pl.BlockSpec((1, CR), lambda b: (0, 0)),
            pl.BlockSpec((CR, CR), lambda b: (0, 0)),
            pl.BlockSpec((CR, C), lambda b: (0, 0)),
            pl.BlockSpec((9, C), lambda b: (0, 0)),
            pl.BlockSpec((1, C), lambda b: (0, 0)),
            pl.BlockSpec((CR, C), lambda b: (0, 0)),
            pl.BlockSpec((C, C), lambda b: (0, 0)),
            pl.BlockSpec((9, K, K), lambda b: (0, 0, 0)),
            pl.BlockSpec((KB, CR), lambda b: (0, 0)),
            pl.BlockSpec((KB, C), lambda b: (0, 0)),
            pl.BlockSpec((K, K), lambda b: (0, 0)),
        ],
        out_specs=[
            pl.BlockSpec((1, KB, C), lambda b: (b, 0, 0)),
            pl.BlockSpec((1, KB, C), lambda b: (b, 0, 0)),
            pl.BlockSpec((1, 1, KB), lambda b: (b, 0, 0)),
        ],
        out_shape=[
            jax.ShapeDtypeStruct((B, KB, C), jnp.float32),
            jax.ShapeDtypeStruct((B, KB, C), jnp.float32),
            jax.ShapeDtypeStruct((B, 1, KB), jnp.float32),
        ],
    )(xk, s_top, c_top, alpha2, kv_wT, kv_b2, ln_g2, ln_b2, k_wT, v_wT,
      cpe_mat, cpe_b2, q_w, proj_wT, SH64, kmaskT, vmask, eyeK)


def _stage_d(x, QK, VP, CM, proj_b2, beta2, NH, K, tn=512):
    B, N, C = x.shape
    KB = NH * K

    def body(x_ref, qk_ref, vp_ref, cm_ref, pb_ref, bt_ref, o_ref):
        xs = x_ref[0]
        lg = jnp.dot(xs, qk_ref[0], preferred_element_type=jnp.float32)
        cm = cm_ref[0]
        beta = bt_ref[0, 0]
        parts = []
        for h in range(NH):
            sl = lg[:, h * K:(h + 1) * K]
            ch = cm[:, h * K:(h + 1) * K]
            m1 = jnp.max(sl, axis=1, keepdims=True)
            e1 = jnp.exp(sl - m1)
            p1 = e1 / jnp.sum(e1, axis=1, keepdims=True)
            s2 = sl * (1.0 + ch)
            m2 = jnp.max(s2, axis=1, keepdims=True)
            e2 = jnp.exp(s2 - m2)
            p2 = e2 / jnp.sum(e2, axis=1, keepdims=True)
            parts.append(beta * p1 + (1.0 - beta) * p2)
        attn = jnp.concatenate(parts, axis=1)
        o_ref[0] = jnp.dot(attn, vp_ref[0],
                           preferred_element_type=jnp.float32) + pb_ref[...]

    return pl.pallas_call(
        body,
        grid=(B, N // tn),
        in_specs=[
            pl.BlockSpec((1, tn, C), lambda b, i: (b, i, 0)),
            pl.BlockSpec((1, C, KB), lambda b, i: (b, 0, 0)),
            pl.BlockSpec((1, KB, C), lambda b, i: (b, 0, 0)),
            pl.BlockSpec((1, 1, KB), lambda b, i: (b, 0, 0)),
            pl.BlockSpec((1, C), lambda b, i: (0, 0)),
            pl.BlockSpec((1, 1), lambda b, i: (0, 0)),
        ],
        out_specs=pl.BlockSpec((1, tn, C), lambda b, i: (b, i, 0)),
        out_shape=jax.ShapeDtypeStruct((B, N, C), jnp.float32),
    )(x, QK, VP, CM, proj_b2, beta2)


def kernel(x, H, W, cc_w, g_w1, g_b1, g_w2, g_b2, q_w, kv_w, kv_b, ln_g, ln_b,
           k_w, v_w, cpe_w, cpe_b, proj_w, proj_b, alpha, beta):
    B, N, C = x.shape
    Hc = int(math.isqrt(N))
    Wc = N // Hc
    NH = 8
    CR = q_w.shape[0]
    qk_scale = ((C // NH) * 0.5) ** -0.5
    time_level = max(2, max(int(math.log(Hc // 16, 4)), int(math.log(Wc // 16, 4))))
    sc = 4 ** time_level
    K = (Hc // sc) * (Wc // sc)
    Hs = Hc // sc

    cc_mat = cc_w.reshape(C, 9) / C
    cpe_mat = cpe_w.reshape(C, 9).T
    SH = jnp.stack([jnp.eye(Hc, k=o, dtype=jnp.float32) for o in (-1, 0, 1)])
    SHT = jnp.stack([jnp.eye(Wc, k=o, dtype=jnp.float32).T for o in (-1, 0, 1)])
    A8 = [jnp.eye(Hs, k=o, dtype=jnp.float32) for o in (-1, 0, 1)]
    SH64 = jnp.stack([jnp.kron(A8[di], A8[dj])
                      for di in range(3) for dj in range(3)])
    hk = jnp.arange(NH * K) // K
    kmaskT = (hk[:, None] == (jnp.arange(CR) // (CR // NH))[None, :]).astype(jnp.float32)
    vmask = (hk[:, None] == (jnp.arange(C) // (C // NH))[None, :]).astype(jnp.float32)
    eyeK = jnp.eye(K, dtype=jnp.float32)

    gate, M = _stage_a(x, g_w1.T, g_b1.reshape(1, CR), g_w2.T,
                       g_b2.reshape(1, 1), cc_mat)
    M9 = M.transpose(0, 2, 1).reshape(B, 9, Hc, Wc)
    gate_img = gate.reshape(B, Hc, Wc)

    gidx, s_top, c_top = _stage_b(M9, gate_img, SH, SHT, K)

    xk = _sc_gather(x.reshape(B * N, C), gidx.reshape(B * K)).reshape(B, K, C)

    QKT, VP, CM = _stage_c(
        xk, s_top, c_top, alpha.reshape(1, 1), kv_w.T, kv_b.reshape(1, CR),
        ln_g.reshape(1, CR), ln_b.reshape(1, CR), k_w.T, v_w.T, cpe_mat,
        cpe_b.reshape(1, C), q_w, proj_w.T, SH64, kmaskT, vmask, eyeK,
        qk_scale, NH, K)

    out = _stage_d(x, QKT.transpose(0, 2, 1), VP, CM,
                   proj_b.reshape(1, C), beta.reshape(1, 1), NH, K)
    return out


# speedup vs baseline: 2.1784x; 2.1784x over previous
import functools
import math

import jax
import jax.numpy as jnp
from jax import lax
from jax.experimental import pallas as pl
from jax.experimental.pallas import tpu as pltpu
from jax.experimental.pallas import tpu_sc as plsc


def _stage_a(x, g_w1T, g_b1, g_w2T, g_b2, cc_mat, tn=2048):
    B, N, C = x.shape
    CR = g_w1T.shape[1]

    def body(x_ref, w1_ref, b1_ref, w2_ref, b2_ref, cc_ref, g_ref, m_ref):
        xs = x_ref[0]
        h = jnp.dot(xs, w1_ref[...], preferred_element_type=jnp.float32) + b1_ref[...]
        h = jnp.maximum(h, 0.0)
        z = jnp.dot(h, w2_ref[...], preferred_element_type=jnp.float32) + b2_ref[...]
        g_ref[0] = 1.0 / (1.0 + jnp.exp(-z))
        m_ref[0] = jnp.dot(xs, cc_ref[...], preferred_element_type=jnp.float32)

    return pl.pallas_call(
        body,
        grid=(B, N // tn),
        in_specs=[
            pl.BlockSpec((1, tn, C), lambda b, i: (b, i, 0)),
            pl.BlockSpec((C, CR), lambda b, i: (0, 0)),
            pl.BlockSpec((1, CR), lambda b, i: (0, 0)),
            pl.BlockSpec((CR, 1), lambda b, i: (0, 0)),
            pl.BlockSpec((1, 1), lambda b, i: (0, 0)),
            pl.BlockSpec((C, 9), lambda b, i: (0, 0)),
        ],
        out_specs=[
            pl.BlockSpec((1, tn, 1), lambda b, i: (b, i, 0)),
            pl.BlockSpec((1, tn, 9), lambda b, i: (b, i, 0)),
        ],
        out_shape=[
            jax.ShapeDtypeStruct((B, N, 1), jnp.float32),
            jax.ShapeDtypeStruct((B, N, 9), jnp.float32),
        ],
    )(x, g_w1T, g_b1, g_w2T, g_b2, cc_mat)


def _stage_b(M9, gate_img, SH, SHT, K):
    B, _, Hc, Wc = M9.shape
    N = Hc * Wc

    def body(m_ref, g_ref, sh_ref, sht_ref, idx_ref, sv_ref, cv_ref):
        y = jnp.zeros((Hc, Wc), jnp.float32)
        for di in range(3):
            acc = jnp.zeros((Hc, Wc), jnp.float32)
            for dj in range(3):
                acc = acc + jnp.dot(m_ref[0, di * 3 + dj], sht_ref[dj],
                                    preferred_element_type=jnp.float32)
            y = y + jnp.dot(sh_ref[di], acc, preferred_element_type=jnp.float32)
        s1 = jnp.sum(y, axis=1, keepdims=True)
        mu = jnp.sum(s1, axis=0, keepdims=True) / N
        d = y - mu
        s2 = jnp.sum(d * d, axis=1, keepdims=True)
        var = jnp.sum(s2, axis=0, keepdims=True) / N
        curv = d / jnp.sqrt(var + 1e-5)
        score = (jnp.abs(curv) + g_ref[0]) * 0.5

        rowi = lax.broadcasted_iota(jnp.int32, (Hc, Wc), 0)
        coli = lax.broadcasted_iota(jnp.int32, (Hc, Wc), 1)
        I2 = (rowi * Wc + coli).astype(jnp.float32)
        lane = lax.broadcasted_iota(jnp.int32, (1, K), 1).astype(jnp.float32)

        def it(i, carry):
            sc, ia, va, ca = carry
            mr = jnp.max(sc, axis=1, keepdims=True)
            m = jnp.max(mr, axis=0, keepdims=True)
            cand = jnp.where(sc == m, I2, jnp.float32(1e9))
            ir = jnp.min(cand, axis=1, keepdims=True)
            idx = jnp.min(ir, axis=0, keepdims=True)
            hit = I2 == idx
            cr = jnp.sum(jnp.where(hit, curv, 0.0), axis=1, keepdims=True)
            cval = jnp.sum(cr, axis=0, keepdims=True)
            fi = lane == i.astype(jnp.float32)
            ia = jnp.where(fi, idx, ia)
            va = jnp.where(fi, m, va)
            ca = jnp.where(fi, cval, ca)
            sc = jnp.where(hit, jnp.float32(-3e38), sc)
            return sc, ia, va, ca

        z = jnp.zeros((1, K), jnp.float32)
        _, ia, va, ca = lax.fori_loop(0, K, it, (score, z, z, z))
        b = pl.program_id(0)
        ia = ia + b.astype(jnp.float32) * N
        idx_ref[0] = ia.astype(jnp.int32)
        sv_ref[0] = va
        cv_ref[0] = ca

    return pl.pallas_call(
        body,
        grid=(B,),
        in_specs=[
            pl.BlockSpec((1, 9, Hc, Wc), lambda b: (b, 0, 0, 0)),
            pl.BlockSpec((1, Hc, Wc), lambda b: (b, 0, 0)),
            pl.BlockSpec((3, Hc, Hc), lambda b: (0, 0, 0)),
            pl.BlockSpec((3, Wc, Wc), lambda b: (0, 0, 0)),
        ],
        out_specs=[
            pl.BlockSpec((1, 1, K), lambda b: (b, 0, 0)),
            pl.BlockSpec((1, 1, K), lambda b: (b, 0, 0)),
            pl.BlockSpec((1, 1, K), lambda b: (b, 0, 0)),
        ],
        out_shape=[
            jax.ShapeDtypeStruct((B, 1, K), jnp.int32),
            jax.ShapeDtypeStruct((B, 1, K), jnp.float32),
            jax.ShapeDtypeStruct((B, 1, K), jnp.float32),
        ],
    )(M9, gate_img, SH, SHT)


def _sc_gather(x2d, gidx):
    R = gidx.shape[0]
    C = x2d.shape[1]
    rpw = 8
    nw = R // rpw
    mesh = plsc.VectorSubcoreMesh(core_axis_name="c", subcore_axis_name="s")

    @functools.partial(
        pl.kernel,
        mesh=mesh,
        out_type=jax.ShapeDtypeStruct((R, C), jnp.float32),
        scratch_types=[
            pltpu.VMEM((rpw,), jnp.int32),
            pltpu.VMEM((rpw, C), jnp.float32),
            pltpu.SemaphoreType.DMA,
        ],
    )
    def k(x_hbm, idx_hbm, out_hbm, idx_v, rows_v, sem):
        wid = lax.axis_index("s") * 2 + lax.axis_index("c")

        @pl.when(wid < nw)
        def _():
            base = wid * rpw
            pltpu.sync_copy(idx_hbm.at[pl.ds(base, rpw)], idx_v)
            pltpu.async_copy(x_hbm.at[idx_v], rows_v, sem).wait()
            pltpu.sync_copy(rows_v, out_hbm.at[pl.ds(base, rpw)])

    return k(x2d, gidx)


def _erf(z):
    a1, a2, a3, a4, a5 = 0.254829592, -0.284496736, 1.421413741, -1.453152027, 1.061405429
    p = 0.3275911
    s = jnp.sign(z)
    az = jnp.abs(z)
    t = 1.0 / (1.0 + p * az)
    poly = t * (a1 + t * (a2 + t * (a3 + t * (a4 + t * a5))))
    return s * (1.0 - poly * jnp.exp(-az * az))


def _stage_c(xk, s_top, c_top, alpha2, kv_wT, kv_b2, ln_g2, ln_b2, k_wT, v_wT,
             cpe_mat, cpe_b2, q_w, proj_wT, SH64, kmaskT, vmask, eyeK,
             qk_scale, NH, K):
    B, _, C = xk.shape
    CR = kv_wT.shape[1]
    KB = NH * K

    def body(xk_ref, s_ref, c_ref, a_ref, kvw_ref, kvb_ref, lng_ref, lnb_ref,
             kw_ref, vw_ref, cpem_ref, cpeb_ref, qw_ref, pw_ref, sh_ref,
             km_ref, vm_ref, eye_ref, qkt_ref, vp_ref, cm_ref):
        xs = xk_ref[0]
        kv = jnp.dot(xs, kvw_ref[...], preferred_element_type=jnp.float32) + kvb_ref[...]
        mu = jnp.mean(kv, axis=1, keepdims=True)
        d = kv - mu
        var = jnp.mean(d * d, axis=1, keepdims=True)
        kv = d / jnp.sqrt(var + 1e-5) * lng_ref[...] + lnb_ref[...]
        kv = 0.5 * kv * (1.0 + _erf(kv * (2.0 ** -0.5)))
        k = jnp.dot(kv, kw_ref[...], preferred_element_type=jnp.float32)
        v = jnp.dot(kv, vw_ref[...], preferred_element_type=jnp.float32)
        srow = s_ref[0]
        scol = jnp.sum(eye_ref[...] * srow, axis=1, keepdims=True)
        v = v * scol
        cpem = cpem_ref[...]
        vc = jnp.zeros_like(v)
        for t in range(9):
            vc = vc + jnp.dot(sh_ref[t], v,
                              preferred_element_type=jnp.float32) * cpem[t:t + 1, :]
        v = v + vc + cpeb_ref[...]
        kbt = jnp.concatenate([k] * NH, axis=0) * km_ref[...]
        qkt_ref[0] = jnp.dot(kbt, qw_ref[...],
                             preferred_element_type=jnp.float32) * qk_scale
        vb = jnp.concatenate([v] * NH, axis=0) * vm_ref[...]
        vp_ref[0] = jnp.dot(vb, pw_ref[...], preferred_element_type=jnp.float32)
        crow = c_ref[0]
        cm_ref[0] = jnp.concatenate([crow] * NH, axis=1) * a_ref[0, 0]

    return pl.pallas_call(
        body,
        grid=(B,),
        in_specs=[
            pl.BlockSpec((1, K, C), lambda b: (b, 0, 0)),
            pl.BlockSpec((1, 1, K), lambda b: (b, 0, 0)),
            pl.BlockSpec((1, 1, K), lambda b: (b, 0, 0)),
            pl.BlockSpec((1, 1), lambda b: (0, 0)),
            pl.BlockSpec((C, CR), lambda b: (0, 0)),
            pl.BlockSpec((1, CR), lambda b: (0, 0)),
            pl.BlockSpec((1, CR), lambda b: (0, 0)),
            pl.BlockSpec((1, CR), lambda b: (0, 0)),
            pl.BlockSpec((CR, CR), lambda b: (0, 0)),
            pl.BlockSpec((CR, C), lambda b: (0, 0)),
            pl.BlockSpec((9, C), lambda b: (0, 0)),
            pl.BlockSpec((1, C), lambda b: (0, 0)),
            pl.BlockSpec((CR, C), lambda b: (0, 0)),
            pl.BlockSpec((C, C), lambda b: (0, 0)),
            pl.BlockSpec((9, K, K), lambda b: (0, 0, 0)),
            pl.BlockSpec((KB, CR), lambda b: (0, 0)),
            pl.BlockSpec((KB, C), lambda b: (0, 0)),
            pl.BlockSpec((K, K), lambda b: (0, 0)),
        ],
        out_specs=[
            pl.BlockSpec((1, KB, C), lambda b: (b, 0, 0)),
            pl.BlockSpec((1, KB, C), lambda b: (b, 0, 0)),
            pl.BlockSpec((1, 1, KB), lambda b: (b, 0, 0)),
        ],
        out_shape=[
            jax.ShapeDtypeStruct((B, KB, C), jnp.float32),
            jax.ShapeDtypeStruct((B, KB, C), jnp.float32),
            jax.ShapeDtypeStruct((B, 1, KB), jnp.float32),
        ],
    )(xk, s_top, c_top, alpha2, kv_wT, kv_b2, ln_g2, ln_b2, k_wT, v_wT,
      cpe_mat, cpe_b2, q_w, proj_wT, SH64, kmaskT, vmask, eyeK)


def _stage_d(x, QK, VP, CM, seg, segT, proj_b2, beta2, NH, K, tn=1024):
    B, N, C = x.shape
    KB = NH * K

    def body(x_ref, qk_ref, vp_ref, cm_ref, sg_ref, sgt_ref, pb_ref, bt_ref,
             o_ref):
        xs = x_ref[0]
        lg = jnp.dot(xs, qk_ref[0], preferred_element_type=jnp.float32)
        cm = cm_ref[0]
        beta = bt_ref[0, 0]
        sg = sg_ref[...]
        sgt = sgt_ref[...]
        m1 = jnp.max(lg, axis=1, keepdims=True)
        e1 = jnp.exp(lg - m1)
        r1 = 1.0 / jnp.dot(e1, sg, preferred_element_type=jnp.float32)
        p1 = e1 * jnp.dot(r1, sgt, preferred_element_type=jnp.float32)
        lm = lg * (1.0 + cm)
        m2 = jnp.max(lm, axis=1, keepdims=True)
        e2 = jnp.exp(lm - m2)
        r2 = 1.0 / jnp.dot(e2, sg, preferred_element_type=jnp.float32)
        p2 = e2 * jnp.dot(r2, sgt, preferred_element_type=jnp.float32)
        attn = beta * p1 + (1.0 - beta) * p2
        o_ref[0] = jnp.dot(attn, vp_ref[0],
                           preferred_element_type=jnp.float32) + pb_ref[...]

    return pl.pallas_call(
        body,
        grid=(B, N // tn),
        in_specs=[
            pl.BlockSpec((1, tn, C), lambda b, i: (b, i, 0)),
            pl.BlockSpec((1, C, KB), lambda b, i: (b, 0, 0)),
            pl.BlockSpec((1, KB, C), lambda b, i: (b, 0, 0)),
            pl.BlockSpec((1, 1, KB), lambda b, i: (b, 0, 0)),
            pl.BlockSpec((KB, NH), lambda b, i: (0, 0)),
            pl.BlockSpec((NH, KB), lambda b, i: (0, 0)),
            pl.BlockSpec((1, C), lambda b, i: (0, 0)),
            pl.BlockSpec((1, 1), lambda b, i: (0, 0)),
        ],
        out_specs=pl.BlockSpec((1, tn, C), lambda b, i: (b, i, 0)),
        out_shape=jax.ShapeDtypeStruct((B, N, C), jnp.float32),
    )(x, QK, VP, CM, seg, segT, proj_b2, beta2)


def kernel(x, H, W, cc_w, g_w1, g_b1, g_w2, g_b2, q_w, kv_w, kv_b, ln_g, ln_b,
           k_w, v_w, cpe_w, cpe_b, proj_w, proj_b, alpha, beta):
    B, N, C = x.shape
    Hc = int(math.isqrt(N))
    Wc = N // Hc
    NH = 8
    CR = q_w.shape[0]
    qk_scale = ((C // NH) * 0.5) ** -0.5
    time_level = max(2, max(int(math.log(Hc // 16, 4)), int(math.log(Wc // 16, 4))))
    sc = 4 ** time_level
    K = (Hc // sc) * (Wc // sc)
    Hs = Hc // sc

    cc_mat = cc_w.reshape(C, 9) / C
    cpe_mat = cpe_w.reshape(C, 9).T
    SH = jnp.stack([jnp.eye(Hc, k=o, dtype=jnp.float32) for o in (-1, 0, 1)])
    SHT = jnp.stack([jnp.eye(Wc, k=o, dtype=jnp.float32).T for o in (-1, 0, 1)])
    A8 = [jnp.eye(Hs, k=o, dtype=jnp.float32) for o in (-1, 0, 1)]
    SH64 = jnp.stack([jnp.kron(A8[di], A8[dj])
                      for di in range(3) for dj in range(3)])
    hk = jnp.arange(NH * K) // K
    kmaskT = (hk[:, None] == (jnp.arange(CR) // (CR // NH))[None, :]).astype(jnp.float32)
    vmask = (hk[:, None] == (jnp.arange(C) // (C // NH))[None, :]).astype(jnp.float32)
    eyeK = jnp.eye(K, dtype=jnp.float32)
    seg = (hk[:, None] == jnp.arange(NH)[None, :]).astype(jnp.float32)

    gate, M = _stage_a(x, g_w1.T, g_b1.reshape(1, CR), g_w2.T,
                       g_b2.reshape(1, 1), cc_mat)
    M9 = M.transpose(0, 2, 1).reshape(B, 9, Hc, Wc)
    gate_img = gate.reshape(B, Hc, Wc)

    gidx, s_top, c_top = _stage_b(M9, gate_img, SH, SHT, K)

    xk = _sc_gather(x.reshape(B * N, C), gidx.reshape(B * K)).reshape(B, K, C)

    QKT, VP, CM = _stage_c(
        xk, s_top, c_top, alpha.reshape(1, 1), kv_w.T, kv_b.reshape(1, CR),
        ln_g.reshape(1, CR), ln_b.reshape(1, CR), k_w.T, v_w.T, cpe_mat,
        cpe_b.reshape(1, C), q_w, proj_w.T, SH64, kmaskT, vmask, eyeK,
        qk_scale, NH, K)

    out = _stage_d(x, QKT.transpose(0, 2, 1), VP, CM, seg, seg.T,
                   proj_b.reshape(1, C), beta.reshape(1, 1), NH, K)
    return out
